# baseline (speedup 1.0000x reference)
import functools

import numpy as np
import jax
import jax.numpy as jnp
from jax import lax
from jax.experimental import pallas as pl
from jax.experimental.pallas import tpu as pltpu
from jax.experimental.pallas import tpu_sc as plsc

_N_SEG = 64
_NC, _NS = 2, 16
_NW = _NC * _NS


def _segment_starts(nT: int) -> np.ndarray:
    t_vec = np.linspace(1, nT, _N_SEG + 1)
    return np.asarray([int(round(x)) - 1 for x in t_vec[:-1]], dtype=np.int32)


@functools.lru_cache(maxsize=None)
def _build(B: int, nT: int, D: int):
    idx = _segment_starts(nT)
    gidx = (np.arange(B, dtype=np.int64)[:, None] * nT + idx[None, :])
    gidx = gidx.reshape(-1).astype(np.int32)
    n_rows = gidx.size
    assert n_rows % _NW == 0
    rpw = n_rows // _NW

    mesh = plsc.VectorSubcoreMesh(
        core_axis_name="c", subcore_axis_name="s",
        num_cores=_NC, num_subcores=_NS)

    @functools.partial(
        pl.kernel, mesh=mesh,
        out_type=jax.ShapeDtypeStruct((n_rows, D), jnp.float32),
        scratch_types=[
            pltpu.VMEM((rpw,), jnp.int32),
            pltpu.VMEM((rpw, D), jnp.float32),
            pltpu.SemaphoreType.DMA,
        ],
    )
    def gather_rows(table_hbm, idx_hbm, out_hbm, idx_v, rows_v, sem):
        wid = lax.axis_index("s") * _NC + lax.axis_index("c")
        base = wid * rpw
        pltpu.sync_copy(idx_hbm.at[pl.ds(base, rpw)], idx_v)
        pltpu.async_copy(table_hbm.at[idx_v], rows_v, sem).wait()
        pltpu.sync_copy(rows_v, out_hbm.at[pl.ds(base, rpw)])

    return gather_rows, gidx, n_rows


def kernel(inp):
    B, nT, D = inp.shape
    gather_rows, gidx, n_rows = _build(B, nT, D)
    out = gather_rows(inp.reshape(B * nT, D), jnp.asarray(gidx))
    return out.reshape(B, _N_SEG, D)
